# baseline (speedup 1.0000x reference)
import functools
import math

import jax
import jax.numpy as jnp
from jax.experimental import pallas as pl
from jax.experimental.pallas import tpu as pltpu
from jax.experimental.pallas import tpu_sc as plsc

_INT_MIN = -2147483648
_INT_MAX = 2147483647
_BLK = 1024
_NB = 10
_NW = 32
_L = 16


def _order_key(s):
    k = jax.lax.bitcast_convert_type(s, jnp.int32)
    return jnp.where(k >= 0, k, k ^ jnp.int32(0x7FFFFFFF))


def _tc_body(n, k_keep, x_ref, p2_ref, w1_ref, b1_ref, w2_ref, w_ref,
             keep_ref, idsb_ref, tabb_ref, xf_ref,
             xcopy, s0s, s1s, s2s, rows3, ids_s):
    b = pl.program_id(0)
    sscr = [s0s, s1s, s2s]

    @pl.when(b < _NB)
    def _scores():
        xb = x_ref[...]
        sa = jax.lax.dot_general(p2_ref[...], xb, (((1,), (1,)), ((), ())),
                                 preferred_element_type=jnp.float32)
        p = p2_ref[0:1, :]
        norm = jnp.sqrt(jnp.sum(p * p))
        st = sa[0:1, :] / (norm + 1e-16)
        sg = sa[1:2, :]
        h = jnp.tanh(jax.lax.dot_general(xb, w1_ref[...],
                                         (((1,), (0,)), ((), ())),
                                         preferred_element_type=jnp.float32)
                     + b1_ref[...])
        sm = jax.lax.dot_general(w2_ref[...], h, (((1,), (1,)), ((), ())),
                                 preferred_element_type=jnp.float32)
        s0s[pl.ds(b, 1), :] = st
        s1s[pl.ds(b, 1), :] = sm
        s2s[pl.ds(b, 1), :] = sg
        xcopy[pl.ds(b * _BLK, _BLK), :] = xb

    @pl.when(b == _NB)
    def _select():
        col8 = (jax.lax.broadcasted_iota(jnp.int32, (16, _BLK), 0) * _BLK
                + jax.lax.broadcasted_iota(jnp.int32, (16, _BLK), 1))
        thresholds, cnt_gts, amaxs, keys_all, tanh_all = [], [], [], [], []
        for sref in sscr:
            s8 = sref[...]
            key = jnp.where(col8 < n, _order_key(s8), jnp.int32(_INT_MIN))
            keys_all.append(key)
            tanh_all.append(jnp.tanh(s8))
            nneg = jnp.sum(jnp.where(key >= 0, 1, 0).astype(jnp.int32))
            use_neg = nneg < k_keep
            lo0 = jnp.where(use_neg, jnp.int32(_INT_MIN), jnp.int32(0))
            hi0 = jnp.where(use_neg, jnp.int32(-1), jnp.int32(_INT_MAX))

            def bisect(_, carry, key=key):
                lo, hi = carry
                span = hi - lo
                mid = lo + (span >> 1) + (span & 1)
                cnt = jnp.sum(jnp.where(key >= mid, 1, 0).astype(jnp.int32))
                ok = cnt >= k_keep
                return (jnp.where(ok, mid, lo), jnp.where(ok, hi, mid - 1))

            t, _ = jax.lax.fori_loop(0, 31, bisect, (lo0, hi0))
            thresholds.append(t)
            cnt_gts.append(jnp.sum(jnp.where(key > t, 1, 0).astype(jnp.int32)))
            mx = jnp.max(key)
            amaxs.append(jnp.min(jnp.where(key == mx, col8,
                                           jnp.int32(_INT_MAX))))

        w0, w1, w2 = w_ref[0], w_ref[1], w_ref[2]
        ws = [w0, w1, w2]
        a0, a1, a2 = amaxs
        th = jnp.float32(0.01)
        eq10 = a1 == a0
        eq20 = a2 == a0
        eq21 = a2 == a1
        spm0 = w0 + jnp.where(eq10, w1, 0.0) + jnp.where(eq20, w2, 0.0)
        spm1 = w1 + jnp.where(eq21, w2, 0.0)
        keeps = [spm0 > th,
                 jnp.logical_and(~eq10, spm1 > th),
                 jnp.logical_and(~eq20, jnp.logical_and(~eq21, w2 > th))]
        sels_raw = [a0, a1, a2]
        sels = [jnp.where(keeps[p], sels_raw[p], jnp.int32(-1))
                for p in range(3)]

        m = [[None] * 3 for _ in range(3)]
        tv = [[None] * 3 for _ in range(3)]
        for i in range(3):
            key, t, cg = keys_all[i], thresholds[i], cnt_gts[i]
            for p in range(3):
                ap = sels_raw[p]
                hit = col8 == ap
                kv = jnp.sum(jnp.where(hit, key, 0).astype(jnp.int32))
                ceb = jnp.sum(jnp.where((key == t) & (col8 < ap), 1, 0)
                              .astype(jnp.int32))
                mm = jnp.logical_or(kv > t,
                                    jnp.logical_and(kv == t, cg + ceb < k_keep))
                m[i][p] = jnp.where(mm, jnp.float32(1.0), jnp.float32(0.0))
                tv[i][p] = jnp.sum(jnp.where(hit, tanh_all[i], 0.0))

        tabs = [[None] * 3 for _ in range(3)]
        for p in range(3):
            cp = ws[0] * tv[0][p] * m[0][p] + ws[1] * tv[1][p] * m[1][p] \
                + ws[2] * tv[2][p] * m[2][p]
            ids_s[p] = sels[p]
            rows3[pl.ds(p, 1), :] = xcopy[pl.ds(sels_raw[p], 1), :] * cp
            for q in range(3):
                tabs[p][q] = ws[0] * m[0][p] * m[0][q] \
                    + ws[1] * m[1][p] * m[1][q] + ws[2] * m[2][p] * m[2][q]
        ids_s[3] = jnp.int32(0)

        coln = jax.lax.broadcasted_iota(jnp.int32, (1, n), 1)
        kv = jnp.zeros((1, n), jnp.float32)
        for p in range(3):
            kv = kv + jnp.where(coln == sels[p], 1.0, 0.0)
        keep_ref[...] = kv

        li = jax.lax.broadcasted_iota(jnp.int32, (1, 3 * _L), 1) // _L
        idsb_ref[...] = jnp.where(li == 0, sels[0],
                                  jnp.where(li == 1, sels[1], sels[2]))
        lt = jax.lax.broadcasted_iota(jnp.int32, (1, 9 * _L), 1) // _L
        tb = jnp.zeros((1, 9 * _L), jnp.float32)
        for p in range(3):
            for q in range(3):
                tb = jnp.where(lt == 3 * p + q, tabs[p][q], tb)
        tabb_ref[...] = tb

    @pl.when(b > _NB)
    def _xf():
        blk = b - _NB - 1
        xf_ref[...] = jnp.zeros(xf_ref.shape, jnp.float32)
        for p in range(3):
            idp = ids_s[p]
            local = idp - blk * _BLK
            @pl.when(jnp.logical_and(idp >= 0,
                                     jnp.logical_and(local >= 0,
                                                     local < _BLK)))
            def _(p=p, local=local):
                xf_ref[pl.ds(local, 1), :] = rows3[pl.ds(p, 1), :]


def _sc_edge_body(e_per, e_total, ei_hbm, ew_hbm, ids_hbm, tab_hbm, out_hbm,
                  src_v, dst_v, ew_v, out_v, ids_v, tab_v, sem):
    wid = jax.lax.axis_index("s") * 2 + jax.lax.axis_index("c")
    base = wid * e_per
    c1 = pltpu.make_async_copy(ei_hbm.at[pl.ds(base, e_per)], src_v, sem)
    c2 = pltpu.make_async_copy(ei_hbm.at[pl.ds(e_total + base, e_per)], dst_v, sem)
    c3 = pltpu.make_async_copy(ew_hbm.at[pl.ds(base, e_per)], ew_v, sem)
    c1.start()
    c2.start()
    c3.start()
    pltpu.sync_copy(ids_hbm, ids_v)
    pltpu.sync_copy(tab_hbm, tab_v)
    c1.wait()
    c2.wait()
    c3.wait()

    id0 = ids_v[pl.ds(0, _L)]
    id1 = ids_v[pl.ds(_L, _L)]
    id2 = ids_v[pl.ds(2 * _L, _L)]
    t = [[tab_v[pl.ds((3 * p + q) * _L, _L)] for q in range(3)]
         for p in range(3)]
    zero = jnp.zeros((_L,), jnp.float32)

    def step(i, carry):
        sl = pl.ds(i * _L, _L)
        sv = src_v[sl]
        dv = dst_v[sl]
        wv = ew_v[sl]
        d0 = dv == id0
        d1 = dv == id1
        d2 = dv == id2
        coef = zero
        for p, idp in enumerate((id0, id1, id2)):
            tp = jnp.where(d0, t[p][0], zero) + jnp.where(d1, t[p][1], zero) \
                + jnp.where(d2, t[p][2], zero)
            coef = coef + jnp.where(sv == idp, tp, zero)
        out_v[sl] = wv * coef
        return carry

    jax.lax.fori_loop(0, e_per // _L, step, 0)
    pltpu.sync_copy(out_v, out_hbm.at[pl.ds(base, e_per)])


@jax.jit
def kernel(x, edge_index, edge_weights, data, batch, mask, weights,
           p_topk, W1, b1, w2, w_gap):
    n, d = x.shape
    e = edge_weights.shape[0]
    k_keep = int(math.ceil(0.5 * n))

    p2 = jnp.stack([p_topk, w_gap])
    b1r = b1.reshape(1, d)
    w2r = w2.reshape(1, d)

    cst = lambda shape: pl.BlockSpec(shape, lambda b: tuple(0 for _ in shape))
    keep, idsb, tabb, x_f = pl.pallas_call(
        functools.partial(_tc_body, n, k_keep),
        grid=(2 * _NB + 1,),
        in_specs=[
            pl.BlockSpec((_BLK, d), lambda b: (jnp.minimum(b, _NB - 1), 0)),
            cst((2, d)), cst((d, d)), cst((1, d)), cst((1, d)),
            pl.BlockSpec(memory_space=pltpu.SMEM),
        ],
        out_specs=[
            cst((1, n)),
            cst((1, 3 * _L)),
            cst((1, 9 * _L)),
            pl.BlockSpec((_BLK, d),
                         lambda b: (jnp.where(b > _NB, b - _NB - 1, 0), 0)),
        ],
        out_shape=[
            jax.ShapeDtypeStruct((1, n), jnp.float32),
            jax.ShapeDtypeStruct((1, 3 * _L), jnp.int32),
            jax.ShapeDtypeStruct((1, 9 * _L), jnp.float32),
            jax.ShapeDtypeStruct((n, d), jnp.float32),
        ],
        scratch_shapes=[
            pltpu.VMEM((_NB * _BLK, d), jnp.float32),
            pltpu.VMEM((16, _BLK), jnp.float32),
            pltpu.VMEM((16, _BLK), jnp.float32),
            pltpu.VMEM((16, _BLK), jnp.float32),
            pltpu.VMEM((8, d), jnp.float32),
            pltpu.SMEM((4,), jnp.int32),
        ],
    )(x, p2, W1, b1r, w2r, weights)

    e_per = e // _NW
    mesh = plsc.VectorSubcoreMesh(core_axis_name="c", subcore_axis_name="s")
    ewf = pl.kernel(
        functools.partial(_sc_edge_body, e_per, e),
        out_type=jax.ShapeDtypeStruct((e,), jnp.float32),
        mesh=mesh,
        scratch_types=[
            pltpu.VMEM((e_per,), jnp.int32),
            pltpu.VMEM((e_per,), jnp.int32),
            pltpu.VMEM((e_per,), jnp.float32),
            pltpu.VMEM((e_per,), jnp.float32),
            pltpu.VMEM((3 * _L,), jnp.int32),
            pltpu.VMEM((9 * _L,), jnp.float32),
            pltpu.SemaphoreType.DMA,
        ],
    )(edge_index.reshape(-1), edge_weights, idsb.reshape(-1), tabb.reshape(-1))

    return (x_f, edge_index, ewf, batch, keep.reshape(n))


# speedup vs baseline: 1.0383x; 1.0383x over previous
import functools
import math

import jax
import jax.numpy as jnp
from jax.experimental import pallas as pl
from jax.experimental.pallas import tpu as pltpu
from jax.experimental.pallas import tpu_sc as plsc

_INT_MIN = -2147483648
_INT_MAX = 2147483647
_BLK = 1024
_NB = 10
_NW = 32
_L = 16


def _order_key(s):
    k = jax.lax.bitcast_convert_type(s, jnp.int32)
    return jnp.where(k >= 0, k, k ^ jnp.int32(0x7FFFFFFF))


def _tc_body(n, k_keep, x_ref, p2_ref, w1_ref, b1_ref, w2_ref, w_ref,
             keep_ref, idsb_ref, tabb_ref, rows3_ref, ids_ref,
             xcopy, s0s, s1s, s2s):
    b = pl.program_id(0)
    sscr = [s0s, s1s, s2s]

    @pl.when(b < _NB)
    def _scores():
        xb = x_ref[...]
        sa = jax.lax.dot_general(p2_ref[...], xb, (((1,), (1,)), ((), ())),
                                 preferred_element_type=jnp.float32)
        p = p2_ref[0:1, :]
        norm = jnp.sqrt(jnp.sum(p * p))
        st = sa[0:1, :] / (norm + 1e-16)
        sg = sa[1:2, :]
        h = jnp.tanh(jax.lax.dot_general(xb, w1_ref[...],
                                         (((1,), (0,)), ((), ())),
                                         preferred_element_type=jnp.float32)
                     + b1_ref[...])
        sm = jax.lax.dot_general(w2_ref[...], h, (((1,), (1,)), ((), ())),
                                 preferred_element_type=jnp.float32)
        s0s[pl.ds(b, 1), :] = st
        s1s[pl.ds(b, 1), :] = sm
        s2s[pl.ds(b, 1), :] = sg
        xcopy[pl.ds(b * _BLK, _BLK), :] = xb

    @pl.when(b == _NB)
    def _select():
        col8 = (jax.lax.broadcasted_iota(jnp.int32, (16, _BLK), 0) * _BLK
                + jax.lax.broadcasted_iota(jnp.int32, (16, _BLK), 1))
        thresholds, cnt_gts, amaxs, keys_all, tanh_all = [], [], [], [], []
        for sref in sscr:
            s8 = sref[...]
            key = jnp.where(col8 < n, _order_key(s8), jnp.int32(_INT_MIN))
            keys_all.append(key)
            tanh_all.append(jnp.tanh(s8))
            nneg = jnp.sum(jnp.where(key >= 0, 1, 0).astype(jnp.int32))
            use_neg = nneg < k_keep
            lo0 = jnp.where(use_neg, jnp.int32(_INT_MIN), jnp.int32(0))
            hi0 = jnp.where(use_neg, jnp.int32(-1), jnp.int32(_INT_MAX))

            def bisect(_, carry, key=key):
                lo, hi = carry
                span = hi - lo
                mid = lo + (span >> 1) + (span & 1)
                cnt = jnp.sum(jnp.where(key >= mid, 1, 0).astype(jnp.int32))
                ok = cnt >= k_keep
                return (jnp.where(ok, mid, lo), jnp.where(ok, hi, mid - 1))

            t, _ = jax.lax.fori_loop(0, 31, bisect, (lo0, hi0))
            thresholds.append(t)
            cnt_gts.append(jnp.sum(jnp.where(key > t, 1, 0).astype(jnp.int32)))
            mx = jnp.max(key)
            amaxs.append(jnp.min(jnp.where(key == mx, col8,
                                           jnp.int32(_INT_MAX))))

        w0, w1, w2 = w_ref[0], w_ref[1], w_ref[2]
        ws = [w0, w1, w2]
        a0, a1, a2 = amaxs
        th = jnp.float32(0.01)
        eq10 = a1 == a0
        eq20 = a2 == a0
        eq21 = a2 == a1
        spm0 = w0 + jnp.where(eq10, w1, 0.0) + jnp.where(eq20, w2, 0.0)
        spm1 = w1 + jnp.where(eq21, w2, 0.0)
        keeps = [spm0 > th,
                 jnp.logical_and(~eq10, spm1 > th),
                 jnp.logical_and(~eq20, jnp.logical_and(~eq21, w2 > th))]
        sels_raw = [a0, a1, a2]
        sels = [jnp.where(keeps[p], sels_raw[p], jnp.int32(-1))
                for p in range(3)]

        m = [[None] * 3 for _ in range(3)]
        tv = [[None] * 3 for _ in range(3)]
        for i in range(3):
            key, t, cg = keys_all[i], thresholds[i], cnt_gts[i]
            for p in range(3):
                ap = sels_raw[p]
                hit = col8 == ap
                kv = jnp.sum(jnp.where(hit, key, 0).astype(jnp.int32))
                ceb = jnp.sum(jnp.where((key == t) & (col8 < ap), 1, 0)
                              .astype(jnp.int32))
                mm = jnp.logical_or(kv > t,
                                    jnp.logical_and(kv == t, cg + ceb < k_keep))
                m[i][p] = jnp.where(mm, jnp.float32(1.0), jnp.float32(0.0))
                tv[i][p] = jnp.sum(jnp.where(hit, tanh_all[i], 0.0))

        tabs = [[None] * 3 for _ in range(3)]
        for p in range(3):
            cp = ws[0] * tv[0][p] * m[0][p] + ws[1] * tv[1][p] * m[1][p] \
                + ws[2] * tv[2][p] * m[2][p]
            ids_ref[p] = sels[p]
            rows3_ref[pl.ds(p, 1), :] = xcopy[pl.ds(sels_raw[p], 1), :] * cp
            for q in range(3):
                tabs[p][q] = ws[0] * m[0][p] * m[0][q] \
                    + ws[1] * m[1][p] * m[1][q] + ws[2] * m[2][p] * m[2][q]
        ids_ref[3] = jnp.int32(0)
        for p in range(3, 8):
            rows3_ref[pl.ds(p, 1), :] = jnp.zeros((1, rows3_ref.shape[1]),
                                                  jnp.float32)

        coln = jax.lax.broadcasted_iota(jnp.int32, (1, n), 1)
        kv = jnp.zeros((1, n), jnp.float32)
        for p in range(3):
            kv = kv + jnp.where(coln == sels[p], 1.0, 0.0)
        keep_ref[...] = kv

        li = jax.lax.broadcasted_iota(jnp.int32, (1, 3 * _L), 1) // _L
        idsb_ref[...] = jnp.where(li == 0, sels[0],
                                  jnp.where(li == 1, sels[1], sels[2]))
        lt = jax.lax.broadcasted_iota(jnp.int32, (1, 9 * _L), 1) // _L
        tb = jnp.zeros((1, 9 * _L), jnp.float32)
        for p in range(3):
            for q in range(3):
                tb = jnp.where(lt == 3 * p + q, tabs[p][q], tb)
        tabb_ref[...] = tb

def _xf_body(rows3_ref, ids_ref, xf_ref):
    b = pl.program_id(0)
    xf_ref[...] = jnp.zeros(xf_ref.shape, jnp.float32)
    for p in range(3):
        idp = ids_ref[p]
        local = idp - b * _BLK
        @pl.when(jnp.logical_and(idp >= 0,
                                 jnp.logical_and(local >= 0, local < _BLK)))
        def _(p=p, local=local):
            xf_ref[pl.ds(local, 1), :] = rows3_ref[pl.ds(p, 1), :]


def _sc_edge_body(e_per, e_total, ei_hbm, ew_hbm, ids_hbm, tab_hbm, out_hbm,
                  src_v, dst_v, ew_v, out_v, ids_v, tab_v, sem):
    wid = jax.lax.axis_index("s") * 2 + jax.lax.axis_index("c")
    base = wid * e_per
    c1 = pltpu.make_async_copy(ei_hbm.at[pl.ds(base, e_per)], src_v, sem)
    c2 = pltpu.make_async_copy(ei_hbm.at[pl.ds(e_total + base, e_per)], dst_v, sem)
    c3 = pltpu.make_async_copy(ew_hbm.at[pl.ds(base, e_per)], ew_v, sem)
    c1.start()
    c2.start()
    c3.start()
    pltpu.sync_copy(ids_hbm, ids_v)
    pltpu.sync_copy(tab_hbm, tab_v)
    c1.wait()
    c2.wait()
    c3.wait()

    id0 = ids_v[pl.ds(0, _L)]
    id1 = ids_v[pl.ds(_L, _L)]
    id2 = ids_v[pl.ds(2 * _L, _L)]
    t = [[tab_v[pl.ds((3 * p + q) * _L, _L)] for q in range(3)]
         for p in range(3)]
    zero = jnp.zeros((_L,), jnp.float32)

    def step(i, carry):
        sl = pl.ds(i * _L, _L)
        sv = src_v[sl]
        dv = dst_v[sl]
        wv = ew_v[sl]
        d0 = dv == id0
        d1 = dv == id1
        d2 = dv == id2
        coef = zero
        for p, idp in enumerate((id0, id1, id2)):
            tp = jnp.where(d0, t[p][0], zero) + jnp.where(d1, t[p][1], zero) \
                + jnp.where(d2, t[p][2], zero)
            coef = coef + jnp.where(sv == idp, tp, zero)
        out_v[sl] = wv * coef
        return carry

    jax.lax.fori_loop(0, e_per // _L, step, 0)
    pltpu.sync_copy(out_v, out_hbm.at[pl.ds(base, e_per)])


@jax.jit
def kernel(x, edge_index, edge_weights, data, batch, mask, weights,
           p_topk, W1, b1, w2, w_gap):
    n, d = x.shape
    e = edge_weights.shape[0]
    k_keep = int(math.ceil(0.5 * n))

    p2 = jnp.stack([p_topk, w_gap])
    b1r = b1.reshape(1, d)
    w2r = w2.reshape(1, d)

    cst = lambda shape: pl.BlockSpec(shape, lambda b: tuple(0 for _ in shape))
    keep, idsb, tabb, rows3, ids = pl.pallas_call(
        functools.partial(_tc_body, n, k_keep),
        grid=(_NB + 1,),
        in_specs=[
            pl.BlockSpec((_BLK, d), lambda b: (jnp.minimum(b, _NB - 1), 0)),
            cst((2, d)), cst((d, d)), cst((1, d)), cst((1, d)),
            pl.BlockSpec(memory_space=pltpu.SMEM),
        ],
        out_specs=[
            cst((1, n)),
            cst((1, 3 * _L)),
            cst((1, 9 * _L)),
            cst((8, d)),
            pl.BlockSpec(memory_space=pltpu.SMEM),
        ],
        out_shape=[
            jax.ShapeDtypeStruct((1, n), jnp.float32),
            jax.ShapeDtypeStruct((1, 3 * _L), jnp.int32),
            jax.ShapeDtypeStruct((1, 9 * _L), jnp.float32),
            jax.ShapeDtypeStruct((8, d), jnp.float32),
            jax.ShapeDtypeStruct((4,), jnp.int32),
        ],
        scratch_shapes=[
            pltpu.VMEM((_NB * _BLK, d), jnp.float32),
            pltpu.VMEM((16, _BLK), jnp.float32),
            pltpu.VMEM((16, _BLK), jnp.float32),
            pltpu.VMEM((16, _BLK), jnp.float32),
        ],
    )(x, p2, W1, b1r, w2r, weights)

    e_per = e // _NW
    mesh = plsc.VectorSubcoreMesh(core_axis_name="c", subcore_axis_name="s")
    ewf = pl.kernel(
        functools.partial(_sc_edge_body, e_per, e),
        out_type=jax.ShapeDtypeStruct((e,), jnp.float32),
        mesh=mesh,
        scratch_types=[
            pltpu.VMEM((e_per,), jnp.int32),
            pltpu.VMEM((e_per,), jnp.int32),
            pltpu.VMEM((e_per,), jnp.float32),
            pltpu.VMEM((e_per,), jnp.float32),
            pltpu.VMEM((3 * _L,), jnp.int32),
            pltpu.VMEM((9 * _L,), jnp.float32),
            pltpu.SemaphoreType.DMA,
        ],
    )(edge_index.reshape(-1), edge_weights, idsb.reshape(-1), tabb.reshape(-1))

    x_f = pl.pallas_call(
        _xf_body,
        grid=(_NB,),
        in_specs=[
            pl.BlockSpec((8, d), lambda b: (0, 0)),
            pl.BlockSpec(memory_space=pltpu.SMEM),
        ],
        out_specs=pl.BlockSpec((_BLK, d), lambda b: (b, 0)),
        out_shape=jax.ShapeDtypeStruct((n, d), jnp.float32),
    )(rows3, ids)

    return (x_f, edge_index, ewf, batch, keep.reshape(n))


# speedup vs baseline: 1.0817x; 1.0419x over previous
import functools
import math

import jax
import jax.numpy as jnp
from jax.experimental import pallas as pl
from jax.experimental.pallas import tpu as pltpu
from jax.experimental.pallas import tpu_sc as plsc

_INT_MIN = -2147483648
_INT_MAX = 2147483647
_CHUNK = 1280
_NW = 32
_L = 16


def _order_key(s):
    k = jax.lax.bitcast_convert_type(s, jnp.int32)
    return jnp.where(k >= 0, k, k ^ jnp.int32(0x7FFFFFFF))


def _fold8(row, npad):
    c = npad // 8
    chunks = [jax.lax.slice(row, (0, i * c), (1, (i + 1) * c)) for i in range(8)]
    return jnp.concatenate(chunks, axis=0)


def _tc_body(n, k_keep, x_ref, p2_ref, w1_ref, b1_ref, w2_ref, w_ref,
             keep_ref, ids_ref, cv_ref, idsb_ref, tabb_ref):
    npad = 8 * _CHUNK

    xall = x_ref[...]
    sa = jax.lax.dot_general(p2_ref[...], xall, (((1,), (1,)), ((), ())),
                             preferred_element_type=jnp.float32)
    p = p2_ref[0:1, :]
    norm = jnp.sqrt(jnp.sum(p * p))
    st = sa[0:1, :] / (norm + 1e-16)
    sg = sa[1:2, :]
    h = jnp.tanh(jax.lax.dot_general(xall, w1_ref[...], (((1,), (0,)), ((), ())),
                                     preferred_element_type=jnp.float32)
                 + b1_ref[...])
    sm = jax.lax.dot_general(w2_ref[...], h, (((1,), (1,)), ((), ())),
                             preferred_element_type=jnp.float32)

    pad = jnp.full((1, npad - n), -jnp.inf, jnp.float32)
    col8 = (jax.lax.broadcasted_iota(jnp.int32, (8, _CHUNK), 0) * _CHUNK
            + jax.lax.broadcasted_iota(jnp.int32, (8, _CHUNK), 1))

    thresholds, cnt_gts, amaxs, keys_all, tanh_all = [], [], [], [], []
    for srow in (st, sm, sg):
        s8 = _fold8(jnp.concatenate([srow, pad], axis=1), npad)
        key = _order_key(s8)
        key = jnp.where(col8 < n, key, jnp.int32(_INT_MIN))
        keys_all.append(key)
        tanh_all.append(jnp.tanh(s8))
        nneg = jnp.sum(jnp.where(key >= 0, 1, 0).astype(jnp.int32))
        use_neg = nneg < k_keep
        lo0 = jnp.where(use_neg, jnp.int32(_INT_MIN), jnp.int32(0))
        hi0 = jnp.where(use_neg, jnp.int32(-1), jnp.int32(_INT_MAX))

        def bisect(_, carry, key=key):
            lo, hi = carry
            span = hi - lo
            mid = lo + (span >> 1) + (span & 1)
            cnt = jnp.sum(jnp.where(key >= mid, 1, 0).astype(jnp.int32))
            ok = cnt >= k_keep
            return (jnp.where(ok, mid, lo), jnp.where(ok, hi, mid - 1))

        t, _ = jax.lax.fori_loop(0, 31, bisect, (lo0, hi0))
        thresholds.append(t)
        cnt_gts.append(jnp.sum(jnp.where(key > t, 1, 0).astype(jnp.int32)))
        mx = jnp.max(key)
        amaxs.append(jnp.min(jnp.where(key == mx, col8, jnp.int32(_INT_MAX))))

    w0, w1, w2 = w_ref[0], w_ref[1], w_ref[2]
    ws = [w0, w1, w2]
    a0, a1, a2 = amaxs
    th = jnp.float32(0.01)
    eq10 = a1 == a0
    eq20 = a2 == a0
    eq21 = a2 == a1
    spm0 = w0 + jnp.where(eq10, w1, 0.0) + jnp.where(eq20, w2, 0.0)
    spm1 = w1 + jnp.where(eq21, w2, 0.0)
    keeps = [spm0 > th,
             jnp.logical_and(~eq10, spm1 > th),
             jnp.logical_and(~eq20, jnp.logical_and(~eq21, w2 > th))]
    sels_raw = [a0, a1, a2]
    sels = [jnp.where(keeps[p], sels_raw[p], jnp.int32(-1)) for p in range(3)]

    m = [[None] * 3 for _ in range(3)]
    tv = [[None] * 3 for _ in range(3)]
    for i in range(3):
        key, t, cg = keys_all[i], thresholds[i], cnt_gts[i]
        for p in range(3):
            ap = sels_raw[p]
            hit = col8 == ap
            kv = jnp.sum(jnp.where(hit, key, 0).astype(jnp.int32))
            ceb = jnp.sum(jnp.where((key == t) & (col8 < ap), 1, 0)
                          .astype(jnp.int32))
            mm = jnp.logical_or(kv > t,
                                jnp.logical_and(kv == t, cg + ceb < k_keep))
            m[i][p] = jnp.where(mm, jnp.float32(1.0), jnp.float32(0.0))
            tv[i][p] = jnp.sum(jnp.where(hit, tanh_all[i], 0.0))

    tabs = [[None] * 3 for _ in range(3)]
    for p in range(3):
        cp = ws[0] * tv[0][p] * m[0][p] + ws[1] * tv[1][p] * m[1][p] \
            + ws[2] * tv[2][p] * m[2][p]
        ids_ref[p] = sels[p]
        cv_ref[p] = cp
        for q in range(3):
            tabs[p][q] = ws[0] * m[0][p] * m[0][q] + ws[1] * m[1][p] * m[1][q] \
                + ws[2] * m[2][p] * m[2][q]
    ids_ref[3] = jnp.int32(0)
    cv_ref[3] = jnp.float32(0.0)

    li = jax.lax.broadcasted_iota(jnp.int32, (1, 3 * 16), 1) // 16
    idsb_ref[...] = jnp.where(li == 0, sels[0],
                              jnp.where(li == 1, sels[1], sels[2]))
    lt = jax.lax.broadcasted_iota(jnp.int32, (1, 9 * 16), 1) // 16
    tb = jnp.zeros((1, 9 * 16), jnp.float32)
    for p in range(3):
        for q in range(3):
            tb = jnp.where(lt == 3 * p + q, tabs[p][q], tb)
    tabb_ref[...] = tb

    coln = jax.lax.broadcasted_iota(jnp.int32, (1, n), 1)
    kv = jnp.zeros((1, n), jnp.float32)
    for p in range(3):
        kv = kv + jnp.where(coln == sels[p], 1.0, 0.0)
    keep_ref[...] = kv


def _xf_body(x_ref, ids_ref, cv_ref, xf_ref):
    xf_ref[...] = jnp.zeros(xf_ref.shape, jnp.float32)
    for p in range(3):
        @pl.when(ids_ref[p] >= 0)
        def _(p=p):
            xf_ref[pl.ds(ids_ref[p], 1), :] = \
                x_ref[pl.ds(ids_ref[p], 1), :] * cv_ref[p]


def _sc_edge_body(e_per, e_total, ei_hbm, ew_hbm, ids_hbm, tab_hbm, out_hbm,
                  src_v, dst_v, ew_v, out_v, ids_v, tab_v, sem):
    wid = jax.lax.axis_index("s") * 2 + jax.lax.axis_index("c")
    base = wid * e_per
    c1 = pltpu.make_async_copy(ei_hbm.at[pl.ds(base, e_per)], src_v, sem)
    c2 = pltpu.make_async_copy(ei_hbm.at[pl.ds(e_total + base, e_per)], dst_v, sem)
    c3 = pltpu.make_async_copy(ew_hbm.at[pl.ds(base, e_per)], ew_v, sem)
    c1.start()
    c2.start()
    c3.start()
    pltpu.sync_copy(ids_hbm, ids_v)
    pltpu.sync_copy(tab_hbm, tab_v)
    c1.wait()
    c2.wait()
    c3.wait()

    id0 = ids_v[pl.ds(0, _L)]
    id1 = ids_v[pl.ds(_L, _L)]
    id2 = ids_v[pl.ds(2 * _L, _L)]
    t = [[tab_v[pl.ds((3 * p + q) * _L, _L)] for q in range(3)] for p in range(3)]
    zero = jnp.zeros((_L,), jnp.float32)

    @plsc.parallel_loop(0, e_per, step=_L, unroll=8)
    def _loop(i):
        sl = pl.ds(i, _L)
        sv = src_v[sl]
        dv = dst_v[sl]
        wv = ew_v[sl]
        d0 = dv == id0
        d1 = dv == id1
        d2 = dv == id2
        coef = zero
        for p, idp in enumerate((id0, id1, id2)):
            tp = jnp.where(d0, t[p][0], zero) + jnp.where(d1, t[p][1], zero) \
                + jnp.where(d2, t[p][2], zero)
            coef = coef + jnp.where(sv == idp, tp, zero)
        out_v[sl] = wv * coef
    pltpu.sync_copy(out_v, out_hbm.at[pl.ds(base, e_per)])


@jax.jit
def kernel(x, edge_index, edge_weights, data, batch, mask, weights,
           p_topk, W1, b1, w2, w_gap):
    n, d = x.shape
    e = edge_weights.shape[0]
    k_keep = int(math.ceil(0.5 * n))

    p2 = jnp.stack([p_topk, w_gap])
    b1r = b1.reshape(1, d)
    w2r = w2.reshape(1, d)

    vm = lambda: pl.BlockSpec(memory_space=pltpu.MemorySpace.VMEM)
    sm = lambda: pl.BlockSpec(memory_space=pltpu.SMEM)
    keep, ids, cv, idsb, tabb = pl.pallas_call(
        functools.partial(_tc_body, n, k_keep),
        in_specs=[vm(), vm(), vm(), vm(), vm(), sm()],
        out_specs=[vm(), sm(), sm(), vm(), vm()],
        out_shape=[
            jax.ShapeDtypeStruct((1, n), jnp.float32),
            jax.ShapeDtypeStruct((4,), jnp.int32),
            jax.ShapeDtypeStruct((4,), jnp.float32),
            jax.ShapeDtypeStruct((1, 3 * _L), jnp.int32),
            jax.ShapeDtypeStruct((1, 9 * _L), jnp.float32),
        ],
    )(x, p2, W1, b1r, w2r, weights)

    e_per = e // _NW
    mesh = plsc.VectorSubcoreMesh(core_axis_name="c", subcore_axis_name="s")
    ewf = pl.kernel(
        functools.partial(_sc_edge_body, e_per, e),
        out_type=jax.ShapeDtypeStruct((e,), jnp.float32),
        mesh=mesh,
        scratch_types=[
            pltpu.VMEM((e_per,), jnp.int32),
            pltpu.VMEM((e_per,), jnp.int32),
            pltpu.VMEM((e_per,), jnp.float32),
            pltpu.VMEM((e_per,), jnp.float32),
            pltpu.VMEM((3 * _L,), jnp.int32),
            pltpu.VMEM((9 * _L,), jnp.float32),
            pltpu.SemaphoreType.DMA,
        ],
    )(edge_index.reshape(-1), edge_weights, idsb.reshape(-1), tabb.reshape(-1))

    x_f = pl.pallas_call(
        _xf_body,
        in_specs=[vm(), sm(), sm()],
        out_specs=vm(),
        out_shape=jax.ShapeDtypeStruct((n, d), jnp.float32),
    )(x, ids, cv)

    return (x_f, edge_index, ewf, batch, keep.reshape(n))


# speedup vs baseline: 1.1176x; 1.0332x over previous
import functools
import math

import jax
import jax.numpy as jnp
from jax.experimental import pallas as pl
from jax.experimental.pallas import tpu as pltpu
from jax.experimental.pallas import tpu_sc as plsc

_INT_MIN = -2147483648
_INT_MAX = 2147483647
_CHUNK = 1280
_NW = 32
_L = 16


def _order_key(s):
    k = jax.lax.bitcast_convert_type(s, jnp.int32)
    return jnp.where(k >= 0, k, k ^ jnp.int32(0x7FFFFFFF))


def _fold8(row, npad):
    c = npad // 8
    chunks = [jax.lax.slice(row, (0, i * c), (1, (i + 1) * c)) for i in range(8)]
    return jnp.concatenate(chunks, axis=0)


def _tc_body(n, k_keep, x_ref, p2_ref, w1_ref, b1_ref, w2_ref, w_ref,
             keep_ref, ids_ref, cv_ref, idsb_ref, tabb_ref):
    npad = 8 * _CHUNK

    xall = x_ref[...]
    sa = jax.lax.dot_general(p2_ref[...], xall, (((1,), (1,)), ((), ())),
                             preferred_element_type=jnp.float32)
    p = p2_ref[0:1, :]
    norm = jnp.sqrt(jnp.sum(p * p))
    st = sa[0:1, :] / (norm + 1e-16)
    sg = sa[1:2, :]
    h = jnp.tanh(jax.lax.dot_general(xall, w1_ref[...], (((1,), (0,)), ((), ())),
                                     preferred_element_type=jnp.float32)
                 + b1_ref[...])
    sm = jax.lax.dot_general(w2_ref[...], h, (((1,), (1,)), ((), ())),
                             preferred_element_type=jnp.float32)

    pad = jnp.full((1, npad - n), -jnp.inf, jnp.float32)
    col8 = (jax.lax.broadcasted_iota(jnp.int32, (8, _CHUNK), 0) * _CHUNK
            + jax.lax.broadcasted_iota(jnp.int32, (8, _CHUNK), 1))

    thresholds, cnt_gts, amaxs, keys_all, tanh_all = [], [], [], [], []
    for srow in (st, sm, sg):
        s8 = _fold8(jnp.concatenate([srow, pad], axis=1), npad)
        key = _order_key(s8)
        key = jnp.where(col8 < n, key, jnp.int32(_INT_MIN))
        keys_all.append(key)
        tanh_all.append(jnp.tanh(s8))
        nneg = jnp.sum(jnp.where(key >= 0, 1, 0).astype(jnp.int32))
        use_neg = nneg < k_keep
        lo0 = jnp.where(use_neg, jnp.int32(_INT_MIN), jnp.int32(0))
        hi0 = jnp.where(use_neg, jnp.int32(-1), jnp.int32(_INT_MAX))

        def bisect(_, carry, key=key):
            lo, hi = carry
            span = hi - lo
            mid = lo + (span >> 1) + (span & 1)
            cnt = jnp.sum(jnp.where(key >= mid, 1, 0).astype(jnp.int32))
            ok = cnt >= k_keep
            return (jnp.where(ok, mid, lo), jnp.where(ok, hi, mid - 1))

        t, _ = jax.lax.fori_loop(0, 31, bisect, (lo0, hi0))
        thresholds.append(t)
        cnt_gts.append(jnp.sum(jnp.where(key > t, 1, 0).astype(jnp.int32)))
        mx = jnp.max(key)
        amaxs.append(jnp.min(jnp.where(key == mx, col8, jnp.int32(_INT_MAX))))

    w0, w1, w2 = w_ref[0], w_ref[1], w_ref[2]
    ws = [w0, w1, w2]
    a0, a1, a2 = amaxs
    th = jnp.float32(0.01)
    eq10 = a1 == a0
    eq20 = a2 == a0
    eq21 = a2 == a1
    spm0 = w0 + jnp.where(eq10, w1, 0.0) + jnp.where(eq20, w2, 0.0)
    spm1 = w1 + jnp.where(eq21, w2, 0.0)
    keeps = [spm0 > th,
             jnp.logical_and(~eq10, spm1 > th),
             jnp.logical_and(~eq20, jnp.logical_and(~eq21, w2 > th))]
    sels_raw = [a0, a1, a2]
    sels = [jnp.where(keeps[p], sels_raw[p], jnp.int32(-1)) for p in range(3)]

    m = [[None] * 3 for _ in range(3)]
    tv = [[None] * 3 for _ in range(3)]
    for i in range(3):
        key, t, cg = keys_all[i], thresholds[i], cnt_gts[i]
        for p in range(3):
            ap = sels_raw[p]
            hit = col8 == ap
            kv = jnp.sum(jnp.where(hit, key, 0).astype(jnp.int32))
            ceb = jnp.sum(jnp.where((key == t) & (col8 < ap), 1, 0)
                          .astype(jnp.int32))
            mm = jnp.logical_or(kv > t,
                                jnp.logical_and(kv == t, cg + ceb < k_keep))
            m[i][p] = jnp.where(mm, jnp.float32(1.0), jnp.float32(0.0))
            tv[i][p] = jnp.sum(jnp.where(hit, tanh_all[i], 0.0))

    tabs = [[None] * 3 for _ in range(3)]
    for p in range(3):
        cp = ws[0] * tv[0][p] * m[0][p] + ws[1] * tv[1][p] * m[1][p] \
            + ws[2] * tv[2][p] * m[2][p]
        ids_ref[p] = sels[p]
        cv_ref[p] = cp
        for q in range(3):
            tabs[p][q] = ws[0] * m[0][p] * m[0][q] + ws[1] * m[1][p] * m[1][q] \
                + ws[2] * m[2][p] * m[2][q]
    ids_ref[3] = jnp.int32(0)
    cv_ref[3] = jnp.float32(0.0)

    li = jax.lax.broadcasted_iota(jnp.int32, (1, 3 * 16), 1) // 16
    idsb_ref[...] = jnp.where(li == 0, sels[0],
                              jnp.where(li == 1, sels[1], sels[2]))
    lt = jax.lax.broadcasted_iota(jnp.int32, (1, 9 * 16), 1) // 16
    tb = jnp.zeros((1, 9 * 16), jnp.float32)
    for p in range(3):
        for q in range(3):
            tb = jnp.where(lt == 3 * p + q, tabs[p][q], tb)
    tabb_ref[...] = tb

    coln = jax.lax.broadcasted_iota(jnp.int32, (1, n), 1)
    kv = jnp.zeros((1, n), jnp.float32)
    for p in range(3):
        kv = kv + jnp.where(coln == sels[p], 1.0, 0.0)
    keep_ref[...] = kv


def _xf_body(x_ref, ids_ref, cv_ref, xf_ref):
    xf_ref[...] = jnp.zeros(xf_ref.shape, jnp.float32)
    for p in range(3):
        @pl.when(ids_ref[p] >= 0)
        def _(p=p):
            xf_ref[pl.ds(ids_ref[p], 1), :] = \
                x_ref[pl.ds(ids_ref[p], 1), :] * cv_ref[p]


def _sc_edge_body(e_per, e_total, ei_hbm, ew_hbm, ids_hbm, tab_hbm, out_hbm,
                  src_v, dst_v, ew_v, out_v, ids_v, tab_v, sem):
    wid = jax.lax.axis_index("s") * 2 + jax.lax.axis_index("c")
    base = wid * e_per
    c1 = pltpu.make_async_copy(ei_hbm.at[pl.ds(base, e_per)], src_v, sem)
    c2 = pltpu.make_async_copy(ei_hbm.at[pl.ds(e_total + base, e_per)], dst_v, sem)
    c3 = pltpu.make_async_copy(ew_hbm.at[pl.ds(base, e_per)], ew_v, sem)
    c1.start()
    c2.start()
    c3.start()
    pltpu.sync_copy(ids_hbm, ids_v)
    pltpu.sync_copy(tab_hbm, tab_v)
    c1.wait()
    c2.wait()
    c3.wait()

    id0 = ids_v[pl.ds(0, _L)]
    id1 = ids_v[pl.ds(_L, _L)]
    id2 = ids_v[pl.ds(2 * _L, _L)]
    t = [[tab_v[pl.ds((3 * p + q) * _L, _L)] for q in range(3)] for p in range(3)]
    zero = jnp.zeros((_L,), jnp.float32)

    def step(i, carry):
        sl = pl.ds(i * _L, _L)
        sv = src_v[sl]
        dv = dst_v[sl]
        wv = ew_v[sl]
        d0 = dv == id0
        d1 = dv == id1
        d2 = dv == id2
        coef = zero
        for p, idp in enumerate((id0, id1, id2)):
            tp = jnp.where(d0, t[p][0], zero) \
                + jnp.where(d1, t[p][1], zero) \
                + jnp.where(d2, t[p][2], zero)
            coef = coef + jnp.where(sv == idp, tp, zero)
        out_v[sl] = wv * coef
        return carry

    jax.lax.fori_loop(0, e_per // _L, step, 0)
    pltpu.sync_copy(out_v, out_hbm.at[pl.ds(base, e_per)])


@jax.jit
def kernel(x, edge_index, edge_weights, data, batch, mask, weights,
           p_topk, W1, b1, w2, w_gap):
    n, d = x.shape
    e = edge_weights.shape[0]
    k_keep = int(math.ceil(0.5 * n))

    p2 = jnp.stack([p_topk, w_gap])
    b1r = b1.reshape(1, d)
    w2r = w2.reshape(1, d)

    vm = lambda: pl.BlockSpec(memory_space=pltpu.MemorySpace.VMEM)
    sm = lambda: pl.BlockSpec(memory_space=pltpu.SMEM)
    keep, ids, cv, idsb, tabb = pl.pallas_call(
        functools.partial(_tc_body, n, k_keep),
        in_specs=[vm(), vm(), vm(), vm(), vm(), sm()],
        out_specs=[vm(), sm(), sm(), vm(), vm()],
        out_shape=[
            jax.ShapeDtypeStruct((1, n), jnp.float32),
            jax.ShapeDtypeStruct((4,), jnp.int32),
            jax.ShapeDtypeStruct((4,), jnp.float32),
            jax.ShapeDtypeStruct((1, 3 * _L), jnp.int32),
            jax.ShapeDtypeStruct((1, 9 * _L), jnp.float32),
        ],
    )(x, p2, W1, b1r, w2r, weights)

    e_per = e // _NW
    mesh = plsc.VectorSubcoreMesh(core_axis_name="c", subcore_axis_name="s")
    ewf = pl.kernel(
        functools.partial(_sc_edge_body, e_per, e),
        out_type=jax.ShapeDtypeStruct((e,), jnp.float32),
        mesh=mesh,
        scratch_types=[
            pltpu.VMEM((e_per,), jnp.int32),
            pltpu.VMEM((e_per,), jnp.int32),
            pltpu.VMEM((e_per,), jnp.float32),
            pltpu.VMEM((e_per,), jnp.float32),
            pltpu.VMEM((3 * _L,), jnp.int32),
            pltpu.VMEM((9 * _L,), jnp.float32),
            pltpu.SemaphoreType.DMA,
        ],
    )(edge_index.reshape(-1), edge_weights, idsb.reshape(-1), tabb.reshape(-1))

    x_f = pl.pallas_call(
        _xf_body,
        in_specs=[vm(), sm(), sm()],
        out_specs=vm(),
        out_shape=jax.ShapeDtypeStruct((n, d), jnp.float32),
    )(x, ids, cv)

    return (x_f, edge_index, ewf, batch, keep.reshape(n))
